# baseline (speedup 1.0000x reference)
import functools

import jax
import jax.numpy as jnp
from jax import lax
from jax.experimental import pallas as pl
from jax.experimental.pallas import tpu as pltpu
from jax.experimental.pallas import tpu_sc as plsc

_NUM_WORKERS = 32
_NB = 4096


def _csum_tree(x):
    acc = x[..., 0:8, :]
    for r in range(1, 16):
        acc = acc + x[..., 8 * r:8 * r + 8, :]
    t = [acc[..., s:s + 1, :] for s in range(8)]
    u = [t[s] + t[(s + 4) % 8] for s in range(8)]
    v = [u[s] + u[(s + 2) % 8] for s in range(8)]
    return v[0] + v[1]


def _sums_tc(fxT, fyT):
    b, d, c, n = fxT.shape

    def body(fx_ref, fy_ref, sx_ref, sy_ref):
        sx_ref[0] = _csum_tree(fx_ref[0])[:, 0, :]
        sy_ref[0] = _csum_tree(fy_ref[0])[:, 0, :]

    return pl.pallas_call(
        body,
        grid=(b, n // _NB),
        in_specs=[
            pl.BlockSpec((1, d, c, _NB), lambda i, j: (i, 0, 0, j)),
            pl.BlockSpec((1, d, c, _NB), lambda i, j: (i, 0, 0, j)),
        ],
        out_specs=[
            pl.BlockSpec((1, d, _NB), lambda i, j: (i, 0, j)),
            pl.BlockSpec((1, d, _NB), lambda i, j: (i, 0, j)),
        ],
        out_shape=[
            jax.ShapeDtypeStruct((b, d, n), jnp.float32),
            jax.ShapeDtypeStruct((b, d, n), jnp.float32),
        ],
    )(fxT, fyT)


def _logits_tc(fxT, fyT, px, py):
    b, d, c, n = fxT.shape

    def body(fx_ref, fy_ref, px_ref, py_ref, out_ref):
        x = fx_ref[0]
        y = fy_ref[0]
        pxv = px_ref[0]
        pyv = py_ref[0]
        phx = (x[0] * pxv[0:1, :] + x[1] * pxv[1:2, :]) + x[2] * pxv[2:3, :]
        phy = (y[0] * pyv[0:1, :] + y[1] * pyv[1:2, :]) + y[2] * pyv[2:3, :]
        prod = phx * phy
        out_ref[0, 0] = _csum_tree(prod)[0]

    out = pl.pallas_call(
        body,
        grid=(b, n // _NB),
        in_specs=[
            pl.BlockSpec((1, d, c, _NB), lambda i, j: (i, 0, 0, j)),
            pl.BlockSpec((1, d, c, _NB), lambda i, j: (i, 0, 0, j)),
            pl.BlockSpec((1, d, _NB), lambda i, j: (i, 0, j)),
            pl.BlockSpec((1, d, _NB), lambda i, j: (i, 0, j)),
        ],
        out_specs=pl.BlockSpec((1, 1, _NB), lambda i, j: (i, 0, j)),
        out_shape=jax.ShapeDtypeStruct((b, 1, n), jnp.float32),
    )(fxT, fyT, px, py)
    return out.reshape(b, n)


def _gather_rows_sc(fx_rows, fy_rows, idx):
    R, N = fx_rows.shape
    B, K = idx.shape
    rows_per_w = R // _NUM_WORKERS
    rows_per_b = R // B
    mesh = plsc.VectorSubcoreMesh(core_axis_name="c", subcore_axis_name="s")

    @functools.partial(
        pl.kernel,
        out_type=(
            jax.ShapeDtypeStruct((R, K), jnp.float32),
            jax.ShapeDtypeStruct((R, K), jnp.float32),
        ),
        mesh=mesh,
        compiler_params=pltpu.CompilerParams(needs_layout_passes=False),
        scratch_types=[
            pltpu.VMEM((K,), jnp.int32),
            pltpu.VMEM((N,), jnp.float32),
            pltpu.VMEM((N,), jnp.float32),
            pltpu.VMEM((N,), jnp.float32),
            pltpu.VMEM((N,), jnp.float32),
            pltpu.VMEM((K,), jnp.float32),
            pltpu.VMEM((K,), jnp.float32),
            pltpu.SemaphoreType.DMA,
            pltpu.SemaphoreType.DMA,
            pltpu.SemaphoreType.DMA,
        ],
    )
    def sc_gather(fx_hbm, fy_hbm, idx_hbm, ox_hbm, oy_hbm,
                  idx_v, row0, row1, row2, row3, out0, out1,
                  sem_in, sem_out0, sem_out1):
        rows = (row0, row1, row2, row3)
        outs = (out0, out1)
        sems_out = (sem_out0, sem_out1)
        wid = lax.axis_index("s") * 2 + lax.axis_index("c")
        base = wid * rows_per_w
        b = base // rows_per_b
        pltpu.sync_copy(idx_hbm.at[b], idx_v)

        def gather_row(row_ref, out_ref):
            def g_body(j, _):
                iv = idx_v[pl.ds(j * 16, 16)]
                out_ref[pl.ds(j * 16, 16)] = plsc.load_gather(row_ref, [iv])
                return 0
            lax.fori_loop(0, K // 16, g_body, 0, unroll=8)

        for src_hbm, dst_hbm in ((fx_hbm, ox_hbm), (fy_hbm, oy_hbm)):
            for q in range(3):
                pltpu.async_copy(src_hbm.at[base + q], rows[q], sem_in)

            def quad(p, _, src_hbm=src_hbm, dst_hbm=dst_hbm):
                r0 = base + 4 * p
                for q in range(4):
                    r = r0 + q
                    pltpu.make_async_copy(
                        src_hbm.at[base], rows[q], sem_in).wait()
                    oslot = q % 2

                    @pl.when(4 * p + q >= 2)
                    def _():
                        pltpu.make_async_copy(
                            outs[oslot], dst_hbm.at[r0], sems_out[oslot]).wait()

                    gather_row(rows[q], outs[oslot])
                    pltpu.async_copy(outs[oslot], dst_hbm.at[r], sems_out[oslot])

                    @pl.when(4 * p + q + 3 < rows_per_w)
                    def _():
                        pltpu.async_copy(
                            src_hbm.at[r + 3], rows[(q + 3) % 4], sem_in)
                return 0

            lax.fori_loop(0, rows_per_w // 4, quad, 0)
            pltpu.make_async_copy(outs[0], dst_hbm.at[base], sems_out[0]).wait()
            pltpu.make_async_copy(outs[1], dst_hbm.at[base], sems_out[1]).wait()

    return sc_gather(fx_rows, fy_rows, idx)


def kernel(fx, fy, topk):
    b, c, d, n = fx.shape
    fxT = jnp.transpose(fx, (0, 2, 1, 3))
    fyT = jnp.transpose(fy, (0, 2, 1, 3))

    sx, sy = _sums_tc(fxT, fyT)
    fx_mean = sx * jnp.float32(1.0 / c)
    fy_mean = sy * jnp.float32(1.0 / c)
    fx_par = fx_mean / (jnp.linalg.norm(fx_mean, axis=1)[:, None, :] + 1e-06)
    fy_par = fy_mean / (jnp.linalg.norm(fy_mean, axis=1)[:, None, :] + 1e-06)

    logits = _logits_tc(fxT, fyT, fx_par, fy_par)
    Sc = jax.nn.softmax(logits, axis=-1)
    k = n // 4
    _, idx = jax.lax.top_k(Sc, k)

    fx_rows = fxT.reshape(b * c * d, n)
    fy_rows = fyT.reshape(b * c * d, n)
    ox, oy = _gather_rows_sc(fx_rows, fy_rows, idx.astype(jnp.int32))
    ox = ox.reshape(b, d, c, k).transpose(0, 2, 1, 3)
    oy = oy.reshape(b, d, c, k).transpose(0, 2, 1, 3)
    return ox, oy


# speedup vs baseline: 1.1810x; 1.1810x over previous
import functools

import jax
import jax.numpy as jnp
from jax import lax
from jax.experimental import pallas as pl
from jax.experimental.pallas import tpu as pltpu
from jax.experimental.pallas import tpu_sc as plsc

_NUM_WORKERS = 32
_NB = 4096


def _csum_tree(x):
    acc = x[..., 0:8, :]
    for r in range(1, 16):
        acc = acc + x[..., 8 * r:8 * r + 8, :]
    t = [acc[..., s:s + 1, :] for s in range(8)]
    u = [t[s] + t[(s + 4) % 8] for s in range(8)]
    v = [u[s] + u[(s + 2) % 8] for s in range(8)]
    return v[0] + v[1]


def _sums_tc(fxT, fyT):
    b, d, c, n = fxT.shape

    def body(fx_ref, fy_ref, sx_ref, sy_ref):
        sx_ref[0] = _csum_tree(fx_ref[0])[:, 0, :]
        sy_ref[0] = _csum_tree(fy_ref[0])[:, 0, :]

    return pl.pallas_call(
        body,
        grid=(b, n // _NB),
        in_specs=[
            pl.BlockSpec((1, d, c, _NB), lambda i, j: (i, 0, 0, j)),
            pl.BlockSpec((1, d, c, _NB), lambda i, j: (i, 0, 0, j)),
        ],
        out_specs=[
            pl.BlockSpec((1, d, _NB), lambda i, j: (i, 0, j)),
            pl.BlockSpec((1, d, _NB), lambda i, j: (i, 0, j)),
        ],
        out_shape=[
            jax.ShapeDtypeStruct((b, d, n), jnp.float32),
            jax.ShapeDtypeStruct((b, d, n), jnp.float32),
        ],
    )(fxT, fyT)


def _logits_tc(fxT, fyT, px, py):
    b, d, c, n = fxT.shape

    def body(fx_ref, fy_ref, px_ref, py_ref, out_ref):
        x = fx_ref[0]
        y = fy_ref[0]
        pxv = px_ref[0]
        pyv = py_ref[0]
        phx = (x[0] * pxv[0:1, :] + x[1] * pxv[1:2, :]) + x[2] * pxv[2:3, :]
        phy = (y[0] * pyv[0:1, :] + y[1] * pyv[1:2, :]) + y[2] * pyv[2:3, :]
        prod = phx * phy
        out_ref[0, 0] = _csum_tree(prod)[0]

    out = pl.pallas_call(
        body,
        grid=(b, n // _NB),
        in_specs=[
            pl.BlockSpec((1, d, c, _NB), lambda i, j: (i, 0, 0, j)),
            pl.BlockSpec((1, d, c, _NB), lambda i, j: (i, 0, 0, j)),
            pl.BlockSpec((1, d, _NB), lambda i, j: (i, 0, j)),
            pl.BlockSpec((1, d, _NB), lambda i, j: (i, 0, j)),
        ],
        out_specs=pl.BlockSpec((1, 1, _NB), lambda i, j: (i, 0, j)),
        out_shape=jax.ShapeDtypeStruct((b, 1, n), jnp.float32),
    )(fxT, fyT, px, py)
    return out.reshape(b, n)


def _gather_rows_sc(fx_rows, fy_rows, idx):
    R, N = fx_rows.shape
    B, K = idx.shape
    rows_per_w = R // _NUM_WORKERS
    rows_per_b = R // B
    mesh = plsc.VectorSubcoreMesh(core_axis_name="c", subcore_axis_name="s")

    @functools.partial(
        pl.kernel,
        out_type=(
            jax.ShapeDtypeStruct((R, K), jnp.float32),
            jax.ShapeDtypeStruct((R, K), jnp.float32),
        ),
        mesh=mesh,
        compiler_params=pltpu.CompilerParams(needs_layout_passes=False),
        scratch_types=[
            pltpu.VMEM((K,), jnp.int32),
            pltpu.VMEM((N,), jnp.float32),
            pltpu.VMEM((N,), jnp.float32),
            pltpu.VMEM((N,), jnp.float32),
            pltpu.VMEM((N,), jnp.float32),
            pltpu.VMEM((K,), jnp.float32),
            pltpu.VMEM((K,), jnp.float32),
            pltpu.SemaphoreType.DMA,
            pltpu.SemaphoreType.DMA,
            pltpu.SemaphoreType.DMA,
        ],
    )
    def sc_gather(fx_hbm, fy_hbm, idx_hbm, ox_hbm, oy_hbm,
                  idx_v, row0, row1, row2, row3, out0, out1,
                  sem_in, sem_out0, sem_out1):
        rows = (row0, row1, row2, row3)
        outs = (out0, out1)
        sems_out = (sem_out0, sem_out1)
        wid = lax.axis_index("s") * 2 + lax.axis_index("c")
        base = wid * rows_per_w
        b = base // rows_per_b
        pltpu.sync_copy(idx_hbm.at[b], idx_v)

        def gather_row(row_ref, out_ref):
            def g_body(j, _):
                iv = idx_v[pl.ds(j * 16, 16)]
                out_ref[pl.ds(j * 16, 16)] = plsc.load_gather(row_ref, [iv])
                return 0
            lax.fori_loop(0, K // 16, g_body, 0, unroll=8)

        for src_hbm, dst_hbm in ((fx_hbm, ox_hbm), (fy_hbm, oy_hbm)):
            for q in range(3):
                pltpu.async_copy(src_hbm.at[base + q], rows[q], sem_in)

            def quad(p, _, src_hbm=src_hbm, dst_hbm=dst_hbm):
                r0 = base + 4 * p
                for q in range(4):
                    r = r0 + q
                    pltpu.make_async_copy(
                        src_hbm.at[base], rows[q], sem_in).wait()
                    oslot = q % 2

                    @pl.when(4 * p + q >= 2)
                    def _():
                        pltpu.make_async_copy(
                            outs[oslot], dst_hbm.at[r0], sems_out[oslot]).wait()

                    gather_row(rows[q], outs[oslot])
                    pltpu.async_copy(outs[oslot], dst_hbm.at[r], sems_out[oslot])

                    @pl.when(4 * p + q + 3 < rows_per_w)
                    def _():
                        pltpu.async_copy(
                            src_hbm.at[r + 3], rows[(q + 3) % 4], sem_in)
                return 0

            lax.fori_loop(0, rows_per_w // 4, quad, 0)
            pltpu.make_async_copy(outs[0], dst_hbm.at[base], sems_out[0]).wait()
            pltpu.make_async_copy(outs[1], dst_hbm.at[base], sems_out[1]).wait()

    return sc_gather(fx_rows, fy_rows, idx)


def _scores_fused_tc(fxT, fyT):
    b, d, c, n = fxT.shape

    def body(fx_ref, fy_ref, out_ref):
        x = fx_ref[0]
        y = fy_ref[0]
        inv = jnp.float32(1.0 / c)
        mx = _csum_tree(x)[:, 0, :] * inv
        my = _csum_tree(y)[:, 0, :] * inv
        nx = jnp.sqrt((mx[0:1] * mx[0:1] + mx[2:3] * mx[2:3])
                      + mx[1:2] * mx[1:2]) + jnp.float32(1e-6)
        ny = jnp.sqrt((my[0:1] * my[0:1] + my[2:3] * my[2:3])
                      + my[1:2] * my[1:2]) + jnp.float32(1e-6)
        pxv = mx / nx
        pyv = my / ny
        phx = (x[0] * pxv[0:1, :] + x[1] * pxv[1:2, :]) + x[2] * pxv[2:3, :]
        phy = (y[0] * pyv[0:1, :] + y[1] * pyv[1:2, :]) + y[2] * pyv[2:3, :]
        out_ref[0, 0] = _csum_tree(phx * phy)[0]

    out = pl.pallas_call(
        body,
        grid=(b, n // _NB),
        in_specs=[
            pl.BlockSpec((1, d, c, _NB), lambda i, j: (i, 0, 0, j)),
            pl.BlockSpec((1, d, c, _NB), lambda i, j: (i, 0, 0, j)),
        ],
        out_specs=pl.BlockSpec((1, 1, _NB), lambda i, j: (i, 0, j)),
        out_shape=jax.ShapeDtypeStruct((b, 1, n), jnp.float32),
    )(fxT, fyT)
    return out.reshape(b, n)


def kernel(fx, fy, topk):
    b, c, d, n = fx.shape
    fxT = jnp.transpose(fx, (0, 2, 1, 3))
    fyT = jnp.transpose(fy, (0, 2, 1, 3))

    logits = _scores_fused_tc(fxT, fyT)
    Sc = jax.nn.softmax(logits, axis=-1)
    k = n // 4
    _, idx = jax.lax.top_k(Sc, k)

    fx_rows = fxT.reshape(b * c * d, n)
    fy_rows = fyT.reshape(b * c * d, n)
    ox, oy = _gather_rows_sc(fx_rows, fy_rows, idx.astype(jnp.int32))
    ox = ox.reshape(b, d, c, k).transpose(0, 2, 1, 3)
    oy = oy.reshape(b, d, c, k).transpose(0, 2, 1, 3)
    return ox, oy
